# baseline (speedup 1.0000x reference)
import functools
import math

import jax
import jax.numpy as jnp
from jax import lax
from jax.experimental import pallas as pl
from jax.experimental.pallas import tpu as pltpu
from jax.experimental.pallas import tpu_sc as plsc

D = 64
NC, NS, L = 2, 16, 16
NW = NC * NS
IDX_MINOR = 128
K = 4
R = K * IDX_MINOR
SCALE = math.sqrt(D)


def _make_emb(n_rows_per_w: int):
    ng = n_rows_per_w // K
    b_per_w = n_rows_per_w * IDX_MINOR
    assert n_rows_per_w % K == 0 and ng % 2 == 0

    mesh = plsc.VectorSubcoreMesh(core_axis_name="c", subcore_axis_name="s")

    def body(x_hbm, table_hbm, out_hbm, idx_v, buf0, buf1, sem0, sem1):
        wid = lax.axis_index("s") * NC + lax.axis_index("c")
        pltpu.sync_copy(x_hbm.at[wid], idx_v)
        bufs = (buf0, buf1)
        sems = (sem0, sem1)
        out_base = wid * b_per_w

        def fire(grp, b):
            for j in range(K):
                pltpu.async_copy(
                    table_hbm.at[idx_v.at[grp * K + j]],
                    bufs[b].at[pl.ds(j * IDX_MINOR, IDX_MINOR)],
                    sems[b],
                )

        fire(0, 0)
        fire(1, 1)

        @pl.loop(0, ng, step=2)
        def _outer(g):
            for b in range(2):
                grp = g + b
                buf = bufs[b]
                pltpu.make_async_copy(
                    table_hbm.at[pl.ds(0, R)], buf, sems[b]
                ).wait()

                @plsc.parallel_loop(0, R, step=1, unroll=8)
                def _scale(r):
                    for c in range(D // L):
                        sl = pl.ds(c * L, L)
                        buf[r, sl] = buf[r, sl] * SCALE

                pltpu.sync_copy(
                    buf,
                    out_hbm.at[pl.ds(out_base + grp * R, R), pl.ds(0, D)],
                )

                @pl.when(grp + 2 < ng)
                def _next():
                    fire(grp + 2, b)

    kern = pl.kernel(
        body,
        out_type=jax.ShapeDtypeStruct((NW * b_per_w, 2 * D), jnp.float32),
        mesh=mesh,
        scratch_types=[
            pltpu.VMEM((n_rows_per_w, IDX_MINOR), jnp.int32),
            pltpu.VMEM((R, D), jnp.float32),
            pltpu.VMEM((R, D), jnp.float32),
            pltpu.SemaphoreType.DMA,
            pltpu.SemaphoreType.DMA,
        ],
        compiler_params=pltpu.CompilerParams(use_tc_tiling_on_sc=False),
    )
    return kern


_TBLK = 512


def _tc_transpose_body(tt_ref, out_ref):
    out_ref[:, 0:D] = tt_ref[...].T


def _make_tc_repack(voc: int):
    grid = (voc + _TBLK - 1) // _TBLK
    return pl.pallas_call(
        _tc_transpose_body,
        grid=(grid,),
        in_specs=[pl.BlockSpec((D, _TBLK), lambda i: (0, i))],
        out_specs=pl.BlockSpec((_TBLK, 2 * D), lambda i: (i, 0)),
        out_shape=jax.ShapeDtypeStruct((voc, 2 * D), jnp.float32),
    )


@jax.jit
def kernel(x, table):
    n, m = x.shape
    total = n * m
    voc = table.shape[0]
    n_rows_per_w = total // (NW * IDX_MINOR)
    xr = (x.astype(jnp.int32) * 2).reshape(NW, n_rows_per_w, IDX_MINOR)
    tp = _make_tc_repack(voc)(table.T).reshape(2 * voc, D)
    out = _make_emb(n_rows_per_w)(xr, tp)
    return out[:, :D].reshape(n, m, D)


# speedup vs baseline: 1.6586x; 1.6586x over previous
import functools
import math

import jax
import jax.numpy as jnp
from jax import lax
from jax.experimental import pallas as pl
from jax.experimental.pallas import tpu as pltpu
from jax.experimental.pallas import tpu_sc as plsc

D = 64
NC, NS, L = 2, 16, 16
NW = NC * NS
IDX_MINOR = 128
K = 4
R = K * IDX_MINOR
SCALE = math.sqrt(D)


def _make_emb(n_rows_per_w: int):
    ng = n_rows_per_w // K
    b_per_w = n_rows_per_w * IDX_MINOR
    assert n_rows_per_w % K == 0 and ng % 2 == 0

    mesh = plsc.VectorSubcoreMesh(core_axis_name="c", subcore_axis_name="s")

    def body(x_hbm, table_hbm, out_hbm, idx_v, buf0, buf1, sem0, sem1):
        wid = lax.axis_index("s") * NC + lax.axis_index("c")
        pltpu.sync_copy(x_hbm.at[wid], idx_v)
        bufs = (buf0, buf1)
        sems = (sem0, sem1)
        out_base = wid * b_per_w

        def fire(grp, b):
            for j in range(K):
                pltpu.async_copy(
                    table_hbm.at[idx_v.at[grp * K + j]],
                    bufs[b].at[pl.ds(j * IDX_MINOR, IDX_MINOR)],
                    sems[b],
                )

        fire(0, 0)
        fire(1, 1)

        @pl.loop(0, ng, step=2)
        def _outer(g):
            for b in range(2):
                grp = g + b
                buf = bufs[b]
                pltpu.make_async_copy(
                    table_hbm.at[pl.ds(0, R)], buf, sems[b]
                ).wait()

                @plsc.parallel_loop(0, R, step=1, unroll=8)
                def _scale(r):
                    for c in range(D // L):
                        sl = pl.ds(c * L, L)
                        buf[r, sl] = buf[r, sl] * SCALE

                pltpu.sync_copy(
                    buf,
                    out_hbm.at[pl.ds(out_base + grp * R, R), pl.ds(0, D)],
                )

                @pl.when(grp + 2 < ng)
                def _next():
                    fire(grp + 2, b)

    kern = pl.kernel(
        body,
        out_type=jax.ShapeDtypeStruct((NW * b_per_w, 2 * D), jnp.float32),
        mesh=mesh,
        scratch_types=[
            pltpu.VMEM((n_rows_per_w, IDX_MINOR), jnp.int32),
            pltpu.VMEM((R, D), jnp.float32),
            pltpu.VMEM((R, D), jnp.float32),
            pltpu.SemaphoreType.DMA,
            pltpu.SemaphoreType.DMA,
        ],
        compiler_params=pltpu.CompilerParams(use_tc_tiling_on_sc=False),
    )
    return kern


@jax.jit
def kernel(x, table):
    n, m = x.shape
    total = n * m
    n_rows_per_w = total // (NW * IDX_MINOR)
    xr = x.astype(jnp.int32).reshape(NW, n_rows_per_w, IDX_MINOR)
    out = _make_emb(n_rows_per_w)(xr, table)
    return out[:, :D].reshape(n, m, D)
